# baseline (speedup 1.0000x reference)
import functools

import jax
import jax.numpy as jnp
from jax.experimental import pallas as pl
from jax.experimental.pallas import tpu as pltpu

_N = 20000
_G = 100
_GP = 104
_NB = 2000
_EPS = 1e-8


def _loss_body(nblocks, gt_ref, box_ref, sl_ref, out_ref, bs_ref, acc_ref):
    i = pl.program_id(0)

    @pl.when(i == 0)
    def _init():
        bs_ref[...] = jnp.full((_GP, 1), -1.0, jnp.float32)
        acc_ref[...] = jnp.zeros((1, 1), jnp.float32)

    bt = box_ref[...].T
    st = sl_ref[...].T
    g = gt_ref[...]
    x1b, y1b, x2b, y2b = bt[0:1, :], bt[1:2, :], bt[2:3, :], bt[3:4, :]
    sc, lb = st[0:1, :], st[1:2, :]
    x1g, y1g, x2g, y2g = g[:, 0:1], g[:, 1:2], g[:, 2:3], g[:, 3:4]
    lg = g[:, 4:5]

    wx = jnp.maximum(jnp.minimum(x2g, x2b) - jnp.maximum(x1g, x1b), 0.0)
    wy = jnp.maximum(jnp.minimum(y2g, y2b) - jnp.maximum(y1g, y1b), 0.0)
    inter = wx * wy
    ag = (x2g - x1g) * (y2g - y1g)
    ab = (x2b - x1b) * (y2b - y1b)
    match = (3.0 * inter > ag + ab) & (lg == lb)

    ms = jnp.where(match, sc, -1.0)
    bmax = jnp.max(ms, axis=1, keepdims=True)
    bs_ref[...] = jnp.maximum(bs_ref[...], bmax)
    acc_ref[...] += jnp.sum(jnp.log(1.0 - sc + _EPS), axis=1, keepdims=True)

    @pl.when(i == nblocks - 1)
    def _fin():
        bs = bs_ref[...]
        vf = jnp.where(bs > 0.0, 1.0, 0.0)
        delta = vf * (jnp.log(bs + _EPS) - jnp.log(1.0 - bs + _EPS))
        bsT = bs.reshape(1, _GP)
        vfT = vf.reshape(1, _GP)
        gi = jax.lax.broadcasted_iota(jnp.int32, (_GP, 1), 0)
        giT = jax.lax.broadcasted_iota(jnp.int32, (1, _GP), 1)
        dup = jnp.any((bs == bsT) & (giT < gi) & (vfT > 0.5), axis=1, keepdims=True)
        corr = jnp.sum(jnp.where(dup, 0.0, delta), axis=0, keepdims=True)
        out_ref[...] = -(acc_ref[...] + corr) / _N


@jax.jit
def kernel(gt_bboxes, gt_labels, nms_scores, sorted_labels, sorted_cls_bboxes):
    nblocks = _N // _NB

    gt = jnp.pad(
        jnp.concatenate([gt_bboxes[0].astype(jnp.float32),
                         gt_labels[0].astype(jnp.float32)[:, None]], axis=1),
        ((0, _GP - _G), (0, 3)), constant_values=-1.0)

    sl = jnp.stack([nms_scores.astype(jnp.float32),
                    sorted_labels.astype(jnp.float32)], axis=1)

    out = pl.pallas_call(
        functools.partial(_loss_body, nblocks),
        grid=(nblocks,),
        in_specs=[
            pl.BlockSpec((_GP, 8), lambda i: (0, 0)),
            pl.BlockSpec((_NB, 4), lambda i: (i, 0)),
            pl.BlockSpec((_NB, 2), lambda i: (i, 0)),
        ],
        out_specs=pl.BlockSpec((1, 1), lambda i: (0, 0)),
        out_shape=jax.ShapeDtypeStruct((1, 1), jnp.float32),
        scratch_shapes=[
            pltpu.VMEM((_GP, 1), jnp.float32),
            pltpu.VMEM((1, 1), jnp.float32),
        ],
    )(gt, sorted_cls_bboxes, sl)
    return out[0, 0]


# speedup vs baseline: 1.3793x; 1.3793x over previous
import jax
import jax.numpy as jnp
from jax.experimental import pallas as pl

_N = 20000
_G = 100
_CH = 2048
_EPS = 1e-8


def _loss_body(gt_ref, gl_ref, box_ref, sc_ref, lab_ref, out_ref):
    g = gt_ref[...][0]
    x1g, y1g, x2g, y2g = g[:, 0:1], g[:, 1:2], g[:, 2:3], g[:, 3:4]
    lg = gl_ref[...].astype(jnp.float32).reshape(_G, 1)
    ag = (x2g - x1g) * (y2g - y1g)

    bt = box_ref[...].T
    score_row = sc_ref[...]
    labf = lab_ref[...].astype(jnp.float32)

    best = jnp.full((_G, 1), -1.0, jnp.float32)
    for lo in range(0, _N, _CH):
        hi = min(lo + _CH, _N)
        x1b, y1b = bt[0:1, lo:hi], bt[1:2, lo:hi]
        x2b, y2b = bt[2:3, lo:hi], bt[3:4, lo:hi]
        sc, lb = score_row[:, lo:hi], labf[:, lo:hi]
        wx = jnp.maximum(jnp.minimum(x2g, x2b) - jnp.maximum(x1g, x1b), 0.0)
        wy = jnp.maximum(jnp.minimum(y2g, y2b) - jnp.maximum(y1g, y1b), 0.0)
        inter = wx * wy
        ab = (x2b - x1b) * (y2b - y1b)
        match = (3.0 * inter > ag + ab) & (lg == lb)
        ms = jnp.where(match, sc, -1.0)
        best = jnp.maximum(best, jnp.max(ms, axis=1, keepdims=True))

    base = jnp.sum(jnp.log((1.0 + _EPS) - score_row), axis=1, keepdims=True)

    vf = jnp.where(best > 0.0, 1.0, 0.0)
    delta = vf * (jnp.log(best + _EPS) - jnp.log((1.0 + _EPS) - best))
    bsT = best.reshape(1, _G)
    vfT = vf.reshape(1, _G)
    gi = jax.lax.broadcasted_iota(jnp.int32, (_G, 1), 0)
    giT = jax.lax.broadcasted_iota(jnp.int32, (1, _G), 1)
    dup = jnp.any((best == bsT) & (giT < gi) & (vfT > 0.5), axis=1, keepdims=True)
    corr = jnp.sum(jnp.where(dup, 0.0, delta), axis=0, keepdims=True)
    out_ref[...] = -(base + corr) / _N


@jax.jit
def kernel(gt_bboxes, gt_labels, nms_scores, sorted_labels, sorted_cls_bboxes):
    out = pl.pallas_call(
        _loss_body,
        out_shape=jax.ShapeDtypeStruct((1, 1), jnp.float32),
    )(gt_bboxes.astype(jnp.float32), gt_labels,
      sorted_cls_bboxes.astype(jnp.float32),
      nms_scores.astype(jnp.float32)[None, :], sorted_labels[None, :])
    return out[0, 0]


# speedup vs baseline: 1.5970x; 1.1578x over previous
import jax
import jax.numpy as jnp
from jax.experimental import pallas as pl
from jax.experimental.pallas import tpu as pltpu

_N = 20000
_G = 100
_CH = 512
_EPS = 1e-8


def _loss_body(gt_ref, gl_ref, box_ref, sc_ref, lab_ref, out_ref):
    g = gt_ref[...][0]
    x1g, y1g, x2g, y2g = g[:, 0:1], g[:, 1:2], g[:, 2:3], g[:, 3:4]
    lg = gl_ref[...].astype(jnp.float32).reshape(_G, 1)
    ag = (x2g - x1g) * (y2g - y1g)

    bt = box_ref[...].T
    score_row = sc_ref[...].reshape(1, _N)
    labf = lab_ref[...].astype(jnp.float32).reshape(1, _N)

    best = jnp.full((_G, 1), -1.0, jnp.float32)
    for lo in range(0, _N, _CH):
        hi = min(lo + _CH, _N)
        x1b, y1b = bt[0:1, lo:hi], bt[1:2, lo:hi]
        x2b, y2b = bt[2:3, lo:hi], bt[3:4, lo:hi]
        sc, lb = score_row[:, lo:hi], labf[:, lo:hi]
        wx = jnp.maximum(jnp.minimum(x2g, x2b) - jnp.maximum(x1g, x1b), 0.0)
        wy = jnp.maximum(jnp.minimum(y2g, y2b) - jnp.maximum(y1g, y1b), 0.0)
        inter = wx * wy
        ab = (x2b - x1b) * (y2b - y1b)
        match = (3.0 * inter > ag + ab) & (lg == lb)
        ms = jnp.where(match, sc, -1.0)
        best = jnp.maximum(best, jnp.max(ms, axis=1, keepdims=True))

    base = jnp.sum(jnp.log((1.0 + _EPS) - score_row), axis=1, keepdims=True)

    vf = jnp.where(best > 0.0, 1.0, 0.0)
    delta = vf * (jnp.log(best + _EPS) - jnp.log((1.0 + _EPS) - best))
    bsT = best.reshape(1, _G)
    vfT = vf.reshape(1, _G)
    gi = jax.lax.broadcasted_iota(jnp.int32, (_G, 1), 0)
    giT = jax.lax.broadcasted_iota(jnp.int32, (1, _G), 1)
    dup = jnp.any((best == bsT) & (giT < gi) & (vfT > 0.5), axis=1, keepdims=True)
    corr = jnp.sum(jnp.where(dup, 0.0, delta), axis=0, keepdims=True)
    out_ref[0] = (-(base + corr) / _N)[0, 0]


@jax.jit
def kernel(gt_bboxes, gt_labels, nms_scores, sorted_labels, sorted_cls_bboxes):
    out = pl.pallas_call(
        _loss_body,
        out_shape=jax.ShapeDtypeStruct((1,), jnp.float32),
        out_specs=pl.BlockSpec(memory_space=pltpu.SMEM),
    )(gt_bboxes.astype(jnp.float32), gt_labels,
      sorted_cls_bboxes.astype(jnp.float32),
      nms_scores.astype(jnp.float32), sorted_labels)
    return out[0]


# speedup vs baseline: 1.6192x; 1.0140x over previous
import jax
import jax.numpy as jnp
from jax.experimental import pallas as pl
from jax.experimental.pallas import tpu as pltpu

_N = 20000
_G = 100
_CH = 512
_EPS = 1e-8


def _loss_body(gt_ref, gl_ref, box_ref, sc_ref, lab_ref, out_ref):
    g = gt_ref[...][0]
    x1g, y1g, x2g, y2g = g[:, 0:1], g[:, 1:2], g[:, 2:3], g[:, 3:4]
    lg = gl_ref[...].astype(jnp.float32).reshape(_G, 1)
    ag3 = (x2g - x1g) * (y2g - y1g) * (1.0 / 3.0)

    bt = box_ref[...].T
    score_row = sc_ref[...].reshape(1, _N)
    labf = lab_ref[...].astype(jnp.float32).reshape(1, _N)

    bests = [jnp.full((_G, 1), -1.0, jnp.float32) for _ in range(2)]
    for k, lo in enumerate(range(0, _N, _CH)):
        hi = min(lo + _CH, _N)
        x1b, y1b = bt[0:1, lo:hi], bt[1:2, lo:hi]
        x2b, y2b = bt[2:3, lo:hi], bt[3:4, lo:hi]
        sc, lb = score_row[:, lo:hi], labf[:, lo:hi]
        wx = jnp.maximum(jnp.minimum(x2g, x2b) - jnp.maximum(x1g, x1b), 0.0)
        wy = jnp.maximum(jnp.minimum(y2g, y2b) - jnp.maximum(y1g, y1b), 0.0)
        inter = wx * wy
        ab3 = (x2b - x1b) * (y2b - y1b) * (1.0 / 3.0)
        match = (inter > ag3 + ab3) & (lg == lb)
        ms = jnp.where(match, sc, -1.0)
        bests[k % 2] = jnp.maximum(bests[k % 2], jnp.max(ms, axis=1, keepdims=True))
    best = jnp.maximum(bests[0], bests[1])

    base = jnp.sum(jnp.log((1.0 + _EPS) - score_row), axis=1, keepdims=True)

    vf = jnp.where(best > 0.0, 1.0, 0.0)
    delta = vf * (jnp.log(best + _EPS) - jnp.log((1.0 + _EPS) - best))
    bsT = best.reshape(1, _G)
    vfT = vf.reshape(1, _G)
    gi = jax.lax.broadcasted_iota(jnp.int32, (_G, 1), 0)
    giT = jax.lax.broadcasted_iota(jnp.int32, (1, _G), 1)
    dup = jnp.any((best == bsT) & (giT < gi) & (vfT > 0.5), axis=1, keepdims=True)
    corr = jnp.sum(jnp.where(dup, 0.0, delta), axis=0, keepdims=True)
    out_ref[0] = (-(base + corr) / _N)[0, 0]


@jax.jit
def kernel(gt_bboxes, gt_labels, nms_scores, sorted_labels, sorted_cls_bboxes):
    out = pl.pallas_call(
        _loss_body,
        out_shape=jax.ShapeDtypeStruct((1,), jnp.float32),
        out_specs=pl.BlockSpec(memory_space=pltpu.SMEM),
    )(gt_bboxes.astype(jnp.float32), gt_labels,
      sorted_cls_bboxes.astype(jnp.float32),
      nms_scores.astype(jnp.float32), sorted_labels)
    return out[0]


# speedup vs baseline: 1.6439x; 1.0152x over previous
import jax
import jax.numpy as jnp
from jax.experimental import pallas as pl
from jax.experimental.pallas import tpu as pltpu

_N = 20000
_G = 100
_CH = 512
_EPS = 1e-8
_LOFF = 1024.0


def _loss_body(gt_ref, gl_ref, box_ref, sc_ref, lab_ref, out_ref):
    g = gt_ref[...][0]
    x1g, x2g = g[:, 0:1], g[:, 2:3]
    lg = gl_ref[...].astype(jnp.float32).reshape(_G, 1)
    yoff_g = lg * _LOFF
    y1g = g[:, 1:2] + yoff_g
    y2g = g[:, 3:4] + yoff_g
    ag3 = (x2g - x1g) * (g[:, 3:4] - g[:, 1:2]) * (1.0 / 3.0)

    bt = box_ref[...].T
    score_row = sc_ref[...].reshape(1, _N)
    labf = lab_ref[...].astype(jnp.float32).reshape(1, _N)

    bests = [jnp.full((_G, 1), -1.0, jnp.float32) for _ in range(2)]
    for k, lo in enumerate(range(0, _N, _CH)):
        hi = min(lo + _CH, _N)
        x1b, x2b = bt[0:1, lo:hi], bt[2:3, lo:hi]
        sc = score_row[:, lo:hi]
        yoff_b = labf[:, lo:hi] * _LOFF
        y1b = bt[1:2, lo:hi] + yoff_b
        y2b = bt[3:4, lo:hi] + yoff_b
        wx = jnp.maximum(jnp.minimum(x2g, x2b) - jnp.maximum(x1g, x1b), 0.0)
        wy = jnp.maximum(jnp.minimum(y2g, y2b) - jnp.maximum(y1g, y1b), 0.0)
        inter = wx * wy
        ab3 = (x2b - x1b) * (bt[3:4, lo:hi] - bt[1:2, lo:hi]) * (1.0 / 3.0)
        match = inter > ag3 + ab3
        ms = jnp.where(match, sc, -1.0)
        bests[k % 2] = jnp.maximum(bests[k % 2], jnp.max(ms, axis=1, keepdims=True))
    best = jnp.maximum(bests[0], bests[1])

    base = jnp.sum(jnp.log((1.0 + _EPS) - score_row), axis=1, keepdims=True)

    vf = jnp.where(best > 0.0, 1.0, 0.0)
    delta = vf * (jnp.log(best + _EPS) - jnp.log((1.0 + _EPS) - best))
    bsT = best.reshape(1, _G)
    vfT = vf.reshape(1, _G)
    gi = jax.lax.broadcasted_iota(jnp.int32, (_G, 1), 0)
    giT = jax.lax.broadcasted_iota(jnp.int32, (1, _G), 1)
    dup = jnp.any((best == bsT) & (giT < gi) & (vfT > 0.5), axis=1, keepdims=True)
    corr = jnp.sum(jnp.where(dup, 0.0, delta), axis=0, keepdims=True)
    out_ref[0] = (-(base + corr) / _N)[0, 0]


@jax.jit
def kernel(gt_bboxes, gt_labels, nms_scores, sorted_labels, sorted_cls_bboxes):
    out = pl.pallas_call(
        _loss_body,
        out_shape=jax.ShapeDtypeStruct((1,), jnp.float32),
        out_specs=pl.BlockSpec(memory_space=pltpu.SMEM),
    )(gt_bboxes.astype(jnp.float32), gt_labels,
      sorted_cls_bboxes.astype(jnp.float32),
      nms_scores.astype(jnp.float32), sorted_labels)
    return out[0]


# speedup vs baseline: 1.7056x; 1.0375x over previous
import jax
import jax.numpy as jnp
from jax.experimental import pallas as pl
from jax.experimental.pallas import tpu as pltpu

_N = 20000
_G = 100
_CH = 512
_EPS = 1e-8
_LOFF = 1024.0


def _loss_body(gt_ref, gl_ref, box_ref, sc_ref, lab_ref, out_ref):
    g = gt_ref[...][0]
    x1g, x2g = g[:, 0:1], g[:, 2:3]
    lg = gl_ref[...].astype(jnp.float32).reshape(_G, 1)
    yoff_g = lg * _LOFF
    y1g = g[:, 1:2] + yoff_g
    y2g = g[:, 3:4] + yoff_g
    ag3 = (x2g - x1g) * (g[:, 3:4] - g[:, 1:2]) * (1.0 / 3.0)

    bt = box_ref[...].T
    score_row = sc_ref[...].reshape(1, _N)
    labf = lab_ref[...].astype(jnp.float32).reshape(1, _N)

    bests = [jnp.full((_G, 128), -1.0, jnp.float32) for _ in range(2)]
    for k, lo in enumerate(range(0, _N, _CH)):
        hi = min(lo + _CH, _N)
        x1b, x2b = bt[0:1, lo:hi], bt[2:3, lo:hi]
        sc = score_row[:, lo:hi]
        yoff_b = labf[:, lo:hi] * _LOFF
        y1b = bt[1:2, lo:hi] + yoff_b
        y2b = bt[3:4, lo:hi] + yoff_b
        wx = jnp.maximum(jnp.minimum(x2g, x2b) - jnp.maximum(x1g, x1b), 0.0)
        wy = jnp.maximum(jnp.minimum(y2g, y2b) - jnp.maximum(y1g, y1b), 0.0)
        inter = wx * wy
        ab3 = (x2b - x1b) * (bt[3:4, lo:hi] - bt[1:2, lo:hi]) * (1.0 / 3.0)
        match = inter > ag3 + ab3
        ms = jnp.where(match, sc, -1.0)
        acc = bests[k % 2]
        w = hi - lo
        for j in range(0, w - w % 128, 128):
            acc = jnp.maximum(acc, ms[:, j:j + 128])
        if w % 128:
            acc = jnp.maximum(acc, jnp.max(ms[:, w - w % 128:], axis=1, keepdims=True))
        bests[k % 2] = acc
    best = jnp.max(jnp.maximum(bests[0], bests[1]), axis=1, keepdims=True)

    base = jnp.sum(jnp.log((1.0 + _EPS) - score_row), axis=1, keepdims=True)

    vf = jnp.where(best > 0.0, 1.0, 0.0)
    delta = vf * (jnp.log(best + _EPS) - jnp.log((1.0 + _EPS) - best))
    bsT = best.reshape(1, _G)
    vfT = vf.reshape(1, _G)
    gi = jax.lax.broadcasted_iota(jnp.int32, (_G, 1), 0)
    giT = jax.lax.broadcasted_iota(jnp.int32, (1, _G), 1)
    dup = jnp.any((best == bsT) & (giT < gi) & (vfT > 0.5), axis=1, keepdims=True)
    corr = jnp.sum(jnp.where(dup, 0.0, delta), axis=0, keepdims=True)
    out_ref[0] = (-(base + corr) / _N)[0, 0]


@jax.jit
def kernel(gt_bboxes, gt_labels, nms_scores, sorted_labels, sorted_cls_bboxes):
    out = pl.pallas_call(
        _loss_body,
        out_shape=jax.ShapeDtypeStruct((1,), jnp.float32),
        out_specs=pl.BlockSpec(memory_space=pltpu.SMEM),
    )(gt_bboxes.astype(jnp.float32), gt_labels,
      sorted_cls_bboxes.astype(jnp.float32),
      nms_scores.astype(jnp.float32), sorted_labels)
    return out[0]
